# Initial kernel scaffold; baseline (speedup 1.0000x reference)
import jax
import jax.numpy as jnp
from jax.experimental import pallas as pl


def kernel(X, edge_index, W):
    raise NotImplementedError("write your pallas kernel here")


# speedup vs baseline: 5.4013x; 5.4013x over previous
import functools

import jax
import jax.numpy as jnp
from jax import lax
from jax.experimental import pallas as pl
from jax.experimental.pallas import tpu as pltpu
from jax.experimental.pallas import tpu_sc as plsc

_NC = 2
_NS = 16
_NW = _NC * _NS
_L = 16

_B = 80
_TCB = 1024


def _sc_body(nodes_pad, n_edges,
             x_hbm, s_hbm, row_hbm, col_hbm,
             a_out, t_out, deg_out,
             idx_r, idx_c, cbuf, svals, ones_v, z1d,
             a_sp, t_sp, deg_sp, sem_g, sem_s):
    cid = lax.axis_index("c")
    sid = lax.axis_index("s")
    wid = sid * _NC + cid
    e_per_w = n_edges // _NW
    n_chunks = e_per_w // _B
    per_tile = nodes_pad // _NS

    d_feat = cbuf.shape[1]

    def _fill_row(r, _):
        def _fill_col(c, _):
            cbuf[r, pl.ds(c * _L, _L)] = jnp.zeros((_L,), jnp.float32)
            return 0
        lax.fori_loop(0, d_feat // _L, _fill_col, 0, unroll=8)
        return 0
    lax.fori_loop(0, _B, _fill_row, 0)

    def _fill1(i, _):
        z1d[pl.ds(i * _L, _L)] = jnp.zeros((_L,), jnp.float32)
        return 0
    lax.fori_loop(0, per_tile // _L, _fill1, 0, unroll=4)

    def _fill2(i, _):
        ones_v[pl.ds(i * _L, _L)] = jnp.ones((_L,), jnp.float32)
        return 0
    lax.fori_loop(0, _B // _L, _fill2, 0, unroll=4)

    for j in range(per_tile // _B):
        pltpu.sync_copy(cbuf, a_sp.at[pl.ds(sid * per_tile + j * _B, _B)])
    pltpu.sync_copy(z1d, t_sp.at[pl.ds(sid * per_tile, per_tile)])
    pltpu.sync_copy(z1d, deg_sp.at[pl.ds(sid * per_tile, per_tile)])
    plsc.subcore_barrier()

    base_w = wid * e_per_w

    def _chunk(ci, _):
        base = base_w + ci * _B
        pltpu.sync_copy(row_hbm.at[pl.ds(base, _B)], idx_r)
        pltpu.sync_copy(col_hbm.at[pl.ds(base, _B)], idx_c)
        cp_g = pltpu.async_copy(x_hbm.at[idx_c], cbuf, sem_g)
        cp_s = pltpu.async_copy(s_hbm.at[idx_c], svals, sem_s)
        cp_g.wait()
        cp_s.wait()
        pltpu.sync_copy(cbuf, a_sp.at[idx_r], add=True)
        pltpu.sync_copy(svals, t_sp.at[idx_r], add=True)
        pltpu.sync_copy(ones_v, deg_sp.at[idx_r], add=True)
        return 0

    lax.fori_loop(0, n_chunks, _chunk, 0)
    plsc.subcore_barrier()

    sl = pl.ds(sid * per_tile, per_tile)
    pltpu.sync_copy(a_sp.at[sl], a_out.at[cid, sl])
    pltpu.sync_copy(t_sp.at[sl], t_out.at[cid, sl])
    pltpu.sync_copy(deg_sp.at[sl], deg_out.at[cid, sl])


def _tc_rowsumsq(x_ref, s_ref):
    x = x_ref[...]
    s_ref[...] = jnp.sum(x * x, axis=1)


def _tc_epilogue(x_ref, a_ref, s_ref, t_ref, d_ref, out_ref):
    a = a_ref[0] + a_ref[1]
    dot = jnp.sum(x_ref[...] * a, axis=1)
    t = t_ref[0] + t_ref[1]
    dg = d_ref[0] + d_ref[1]
    sums = dg * s_ref[...] + t - 2.0 * dot
    mean = jnp.where(dg > 0.0, sums / jnp.maximum(dg, 1.0), 0.0)
    out_ref[...] = jnp.tanh(mean)


def kernel(X, edge_index, W):
    del W
    n_nodes, d_feat = X.shape
    n_edges = edge_index.shape[1]
    nodes_pad = ((n_nodes + _NS * _L - 1) // (_NS * _L)) * (_NS * _L)

    Xp = jnp.zeros((nodes_pad, d_feat), X.dtype).at[:n_nodes].set(X)
    grid = nodes_pad // _TCB

    s = pl.pallas_call(
        _tc_rowsumsq,
        grid=(grid,),
        in_specs=[pl.BlockSpec((_TCB, d_feat), lambda i: (i, 0))],
        out_specs=pl.BlockSpec((_TCB,), lambda i: (i,)),
        out_shape=jax.ShapeDtypeStruct((nodes_pad,), jnp.float32),
    )(Xp)

    mesh = plsc.VectorSubcoreMesh(core_axis_name="c", subcore_axis_name="s")
    sc = pl.kernel(
        functools.partial(_sc_body, nodes_pad, n_edges),
        mesh=mesh,
        out_type=(
            jax.ShapeDtypeStruct((_NC, nodes_pad, d_feat), jnp.float32),
            jax.ShapeDtypeStruct((_NC, nodes_pad), jnp.float32),
            jax.ShapeDtypeStruct((_NC, nodes_pad), jnp.float32),
        ),
        scratch_types=[
            pltpu.VMEM((_B,), jnp.int32),
            pltpu.VMEM((_B,), jnp.int32),
            pltpu.VMEM((_B, d_feat), jnp.float32),
            pltpu.VMEM((_B,), jnp.float32),
            pltpu.VMEM((_B,), jnp.float32),
            pltpu.VMEM((nodes_pad // _NS,), jnp.float32),
            pltpu.VMEM_SHARED((nodes_pad, d_feat), jnp.float32),
            pltpu.VMEM_SHARED((nodes_pad,), jnp.float32),
            pltpu.VMEM_SHARED((nodes_pad,), jnp.float32),
            pltpu.SemaphoreType.DMA,
            pltpu.SemaphoreType.DMA,
        ],
    )
    a2, t2, deg2 = sc(Xp, s, edge_index[0], edge_index[1])

    gg_pad = pl.pallas_call(
        _tc_epilogue,
        grid=(grid,),
        in_specs=[
            pl.BlockSpec((_TCB, d_feat), lambda i: (i, 0)),
            pl.BlockSpec((_NC, _TCB, d_feat), lambda i: (0, i, 0)),
            pl.BlockSpec((_TCB,), lambda i: (i,)),
            pl.BlockSpec((_NC, _TCB), lambda i: (0, i)),
            pl.BlockSpec((_NC, _TCB), lambda i: (0, i)),
        ],
        out_specs=pl.BlockSpec((_TCB,), lambda i: (i,)),
        out_shape=jax.ShapeDtypeStruct((nodes_pad,), jnp.float32),
    )(Xp, a2, s, t2, deg2)
    return gg_pad[:n_nodes]


# speedup vs baseline: 8.0223x; 1.4853x over previous
import functools

import jax
import jax.numpy as jnp
from jax import lax
from jax.experimental import pallas as pl
from jax.experimental.pallas import tpu as pltpu
from jax.experimental.pallas import tpu_sc as plsc

_NC = 2
_NS = 16
_NW = _NC * _NS
_L = 16

_B = 80
_TCB = 1024


def _sc_body(nodes_pad, n_edges,
             x_hbm, s_hbm, row_hbm, col_hbm, zb_hbm,
             a_out, t_out, deg_out, *scr):
    idx_r = scr[0:2]
    idx_c = scr[2:4]
    cbuf = scr[4:6]
    sval = scr[6:8]
    ones_v, z1d, a_sp, t_sp, deg_sp = scr[8:13]
    sem_g = scr[13:15]
    sem_s = scr[15:17]

    cid = lax.axis_index("c")
    sid = lax.axis_index("s")
    wid = sid * _NC + cid
    e_per_w = n_edges // _NW
    n_chunks = e_per_w // _B
    per_tile = nodes_pad // _NS

    def _fill1(i, _):
        z1d[pl.ds(i * _L, _L)] = jnp.zeros((_L,), jnp.float32)
        return 0
    lax.fori_loop(0, per_tile // _L, _fill1, 0, unroll=4)

    def _fill2(i, _):
        ones_v[pl.ds(i * _L, _L)] = jnp.ones((_L,), jnp.float32)
        return 0
    lax.fori_loop(0, _B // _L, _fill2, 0, unroll=4)

    pltpu.sync_copy(zb_hbm, a_sp.at[pl.ds(sid * per_tile, per_tile)])
    pltpu.sync_copy(z1d, t_sp.at[pl.ds(sid * per_tile, per_tile)])
    pltpu.sync_copy(z1d, deg_sp.at[pl.ds(sid * per_tile, per_tile)])
    plsc.subcore_barrier()

    base_w = wid * e_per_w

    def _load_idx(b, ci):
        base = base_w + ci * _B
        pltpu.sync_copy(row_hbm.at[pl.ds(base, _B)], idx_r[b])
        pltpu.sync_copy(col_hbm.at[pl.ds(base, _B)], idx_c[b])

    def _issue_gathers(b):
        pltpu.async_copy(x_hbm.at[idx_c[b]], cbuf[b], sem_g[b])
        pltpu.async_copy(s_hbm.at[idx_c[b]], sval[b], sem_s[b])

    def _finish_chunk(b):
        pltpu.make_async_copy(x_hbm.at[idx_c[b]], cbuf[b], sem_g[b]).wait()
        pltpu.make_async_copy(s_hbm.at[idx_c[b]], sval[b], sem_s[b]).wait()
        pltpu.sync_copy(cbuf[b], a_sp.at[idx_r[b]], add=True)
        pltpu.sync_copy(sval[b], t_sp.at[idx_r[b]], add=True)
        pltpu.sync_copy(ones_v, deg_sp.at[idx_r[b]], add=True)

    _load_idx(0, 0)
    _issue_gathers(0)

    def _pair(g, _):
        for b in range(2):
            ci = g * 2 + b
            bn = 1 - b

            @pl.when(ci + 1 < n_chunks)
            def _():
                _load_idx(bn, ci + 1)
                _issue_gathers(bn)

            _finish_chunk(b)
        return 0

    lax.fori_loop(0, n_chunks // 2, _pair, 0)
    if n_chunks % 2:
        _finish_chunk((n_chunks - 1) % 2)
    plsc.subcore_barrier()

    sl = pl.ds(sid * per_tile, per_tile)
    pltpu.sync_copy(a_sp.at[sl], a_out.at[cid, sl])
    pltpu.sync_copy(t_sp.at[sl], t_out.at[cid, sl])
    pltpu.sync_copy(deg_sp.at[sl], deg_out.at[cid, sl])


def _tc_rowsumsq(x_ref, s_ref):
    x = x_ref[...]
    s_ref[...] = jnp.sum(x * x, axis=1)


def _tc_epilogue(x_ref, a_ref, s_ref, t_ref, d_ref, out_ref):
    a = a_ref[0] + a_ref[1]
    dot = jnp.sum(x_ref[...] * a, axis=1)
    t = t_ref[0] + t_ref[1]
    dg = d_ref[0] + d_ref[1]
    sums = dg * s_ref[...] + t - 2.0 * dot
    mean = jnp.where(dg > 0.0, sums / jnp.maximum(dg, 1.0), 0.0)
    out_ref[...] = jnp.tanh(mean)


def kernel(X, edge_index, W):
    del W
    n_nodes, d_feat = X.shape
    n_edges = edge_index.shape[1]
    nodes_pad = ((n_nodes + _NS * _L - 1) // (_NS * _L)) * (_NS * _L)

    Xp = jnp.zeros((nodes_pad, d_feat), X.dtype).at[:n_nodes].set(X)
    grid = nodes_pad // _TCB

    s = pl.pallas_call(
        _tc_rowsumsq,
        grid=(grid,),
        in_specs=[pl.BlockSpec((_TCB, d_feat), lambda i: (i, 0))],
        out_specs=pl.BlockSpec((_TCB,), lambda i: (i,)),
        out_shape=jax.ShapeDtypeStruct((nodes_pad,), jnp.float32),
    )(Xp)

    mesh = plsc.VectorSubcoreMesh(core_axis_name="c", subcore_axis_name="s")
    sc = pl.kernel(
        functools.partial(_sc_body, nodes_pad, n_edges),
        mesh=mesh,
        out_type=(
            jax.ShapeDtypeStruct((_NC, nodes_pad, d_feat), jnp.float32),
            jax.ShapeDtypeStruct((_NC, nodes_pad), jnp.float32),
            jax.ShapeDtypeStruct((_NC, nodes_pad), jnp.float32),
        ),
        scratch_types=[pltpu.VMEM((_B,), jnp.int32) for _ in range(2)]
          + [pltpu.VMEM((_B,), jnp.int32) for _ in range(2)]
          + [pltpu.VMEM((_B, d_feat), jnp.float32) for _ in range(2)]
          + [pltpu.VMEM((_B,), jnp.float32) for _ in range(2)]
          + [
            pltpu.VMEM((_B,), jnp.float32),
            pltpu.VMEM((nodes_pad // _NS,), jnp.float32),
            pltpu.VMEM_SHARED((nodes_pad, d_feat), jnp.float32),
            pltpu.VMEM_SHARED((nodes_pad,), jnp.float32),
            pltpu.VMEM_SHARED((nodes_pad,), jnp.float32),
        ] + [pltpu.SemaphoreType.DMA for _ in range(4)],
    )
    zb = jnp.zeros((nodes_pad // _NS, d_feat), jnp.float32)
    a2, t2, deg2 = sc(Xp, s, edge_index[0], edge_index[1], zb)

    gg_pad = pl.pallas_call(
        _tc_epilogue,
        grid=(grid,),
        in_specs=[
            pl.BlockSpec((_TCB, d_feat), lambda i: (i, 0)),
            pl.BlockSpec((_NC, _TCB, d_feat), lambda i: (0, i, 0)),
            pl.BlockSpec((_TCB,), lambda i: (i,)),
            pl.BlockSpec((_NC, _TCB), lambda i: (0, i)),
            pl.BlockSpec((_NC, _TCB), lambda i: (0, i)),
        ],
        out_specs=pl.BlockSpec((_TCB,), lambda i: (i,)),
        out_shape=jax.ShapeDtypeStruct((nodes_pad,), jnp.float32),
    )(Xp, a2, s, t2, deg2)
    return gg_pad[:n_nodes]


# speedup vs baseline: 10.8921x; 1.3577x over previous
import functools

import jax
import jax.numpy as jnp
from jax import lax
from jax.experimental import pallas as pl
from jax.experimental.pallas import tpu as pltpu
from jax.experimental.pallas import tpu_sc as plsc

_NC = 2
_NS = 16
_NW = _NC * _NS
_L = 16

_B = 80
_TCB = 1024


def _sc_body(nodes_pad, n_edges,
             x_hbm, s_hbm, row_hbm, col_hbm, zb_hbm,
             a_out, t_out, deg_out, *scr):
    idx_rl, idx_cl = scr[0:2]
    idx_r = scr[2:4]
    idx_c = scr[4:6]
    cbuf = scr[6:8]
    sval = scr[8:10]
    ones_v, z1d, a_sp, t_sp, deg_sp = scr[10:15]
    sem_g = scr[15:17]
    sem_s = scr[17:19]

    cid = lax.axis_index("c")
    sid = lax.axis_index("s")
    wid = sid * _NC + cid
    e_per_w = n_edges // _NW
    n_chunks = e_per_w // _B
    per_tile = nodes_pad // _NS

    def _fill2(i, _):
        ones_v[pl.ds(i * _L, _L)] = jnp.ones((_L,), jnp.float32)
        return 0
    lax.fori_loop(0, (_B + _L - 1) // _L, _fill2, 0, unroll=4)

    def _fill1(i, _):
        z1d[pl.ds(i * _L, _L)] = jnp.zeros((_L,), jnp.float32)
        return 0
    lax.fori_loop(0, per_tile // _L, _fill1, 0, unroll=4)

    pltpu.sync_copy(zb_hbm, a_sp.at[pl.ds(sid * per_tile, per_tile)])
    pltpu.sync_copy(z1d, t_sp.at[pl.ds(sid * per_tile, per_tile)])
    pltpu.sync_copy(z1d, deg_sp.at[pl.ds(sid * per_tile, per_tile)])
    plsc.subcore_barrier()

    base_w = wid * e_per_w
    pltpu.sync_copy(row_hbm.at[pl.ds(base_w, e_per_w)], idx_rl)
    pltpu.sync_copy(col_hbm.at[pl.ds(base_w, e_per_w)], idx_cl)

    def _load_idx(b, ci):
        def _mv(k, _):
            sl_d = pl.ds(k * _L, _L)
            sl_s = pl.ds(ci * _B + k * _L, _L)
            idx_r[b][sl_d] = idx_rl[sl_s]
            idx_c[b][sl_d] = idx_cl[sl_s]
            return 0
        lax.fori_loop(0, _B // _L, _mv, 0, unroll=_B // _L)

    def _issue_gathers(b):
        pltpu.async_copy(x_hbm.at[idx_c[b]], cbuf[b], sem_g[b])
        pltpu.async_copy(s_hbm.at[idx_c[b]], sval[b], sem_s[b])

    def _finish_chunk(b):
        pltpu.make_async_copy(x_hbm.at[idx_c[b]], cbuf[b], sem_g[b]).wait()
        pltpu.make_async_copy(s_hbm.at[idx_c[b]], sval[b], sem_s[b]).wait()
        pltpu.sync_copy(cbuf[b], a_sp.at[idx_r[b]], add=True)
        pltpu.sync_copy(sval[b], t_sp.at[idx_r[b]], add=True)
        pltpu.sync_copy(ones_v.at[pl.ds(0, _B)], deg_sp.at[idx_r[b]],
                        add=True)

    _load_idx(0, 0)
    _issue_gathers(0)

    def _pair(g, _):
        for b in range(2):
            ci = g * 2 + b
            bn = 1 - b

            @pl.when(ci + 1 < n_chunks)
            def _():
                _load_idx(bn, ci + 1)
                _issue_gathers(bn)

            _finish_chunk(b)
        return 0

    lax.fori_loop(0, n_chunks // 2, _pair, 0)
    if n_chunks % 2:
        _finish_chunk((n_chunks - 1) % 2)
    plsc.subcore_barrier()

    sl = pl.ds(sid * per_tile, per_tile)
    pltpu.sync_copy(a_sp.at[sl], a_out.at[cid, sl])
    pltpu.sync_copy(t_sp.at[sl], t_out.at[cid, sl])
    pltpu.sync_copy(deg_sp.at[sl], deg_out.at[cid, sl])


def _tc_rowsumsq(x_ref, s_ref):
    x = x_ref[...]
    s_ref[...] = jnp.sum(x * x, axis=1)


def _tc_epilogue(x_ref, a_ref, s_ref, t_ref, d_ref, out_ref):
    a = a_ref[0] + a_ref[1]
    dot = jnp.sum(x_ref[...] * a, axis=1)
    t = t_ref[0] + t_ref[1]
    dg = d_ref[0] + d_ref[1]
    sums = dg * s_ref[...] + t - 2.0 * dot
    mean = jnp.where(dg > 0.0, sums / jnp.maximum(dg, 1.0), 0.0)
    out_ref[...] = jnp.tanh(mean)


def kernel(X, edge_index, W):
    del W
    n_nodes, d_feat = X.shape
    n_edges = edge_index.shape[1]
    nodes_pad = ((n_nodes + _NS * _L - 1) // (_NS * _L)) * (_NS * _L)

    Xp = jnp.zeros((nodes_pad, d_feat), X.dtype).at[:n_nodes].set(X)
    grid = nodes_pad // _TCB

    s = pl.pallas_call(
        _tc_rowsumsq,
        grid=(grid,),
        in_specs=[pl.BlockSpec((_TCB, d_feat), lambda i: (i, 0))],
        out_specs=pl.BlockSpec((_TCB,), lambda i: (i,)),
        out_shape=jax.ShapeDtypeStruct((nodes_pad,), jnp.float32),
    )(Xp)

    mesh = plsc.VectorSubcoreMesh(core_axis_name="c", subcore_axis_name="s")
    sc = pl.kernel(
        functools.partial(_sc_body, nodes_pad, n_edges),
        mesh=mesh,
        out_type=(
            jax.ShapeDtypeStruct((_NC, nodes_pad, d_feat), jnp.float32),
            jax.ShapeDtypeStruct((_NC, nodes_pad), jnp.float32),
            jax.ShapeDtypeStruct((_NC, nodes_pad), jnp.float32),
        ),
        scratch_types=[pltpu.VMEM((n_edges // _NW,), jnp.int32)
                       for _ in range(2)]
          + [pltpu.VMEM((_B,), jnp.int32) for _ in range(2)]
          + [pltpu.VMEM((_B,), jnp.int32) for _ in range(2)]
          + [pltpu.VMEM((_B, d_feat), jnp.float32) for _ in range(2)]
          + [pltpu.VMEM((_B,), jnp.float32) for _ in range(2)]
          + [
            pltpu.VMEM((((_B + _L - 1) // _L) * _L,), jnp.float32),
            pltpu.VMEM((nodes_pad // _NS,), jnp.float32),
            pltpu.VMEM_SHARED((nodes_pad, d_feat), jnp.float32),
            pltpu.VMEM_SHARED((nodes_pad,), jnp.float32),
            pltpu.VMEM_SHARED((nodes_pad,), jnp.float32),
        ] + [pltpu.SemaphoreType.DMA for _ in range(4)],
    )
    zb = jnp.zeros((nodes_pad // _NS, d_feat), jnp.float32)
    a2, t2, deg2 = sc(Xp, s, edge_index[0], edge_index[1], zb)

    gg_pad = pl.pallas_call(
        _tc_epilogue,
        grid=(grid,),
        in_specs=[
            pl.BlockSpec((_TCB, d_feat), lambda i: (i, 0)),
            pl.BlockSpec((_NC, _TCB, d_feat), lambda i: (0, i, 0)),
            pl.BlockSpec((_TCB,), lambda i: (i,)),
            pl.BlockSpec((_NC, _TCB), lambda i: (0, i)),
            pl.BlockSpec((_NC, _TCB), lambda i: (0, i)),
        ],
        out_specs=pl.BlockSpec((_TCB,), lambda i: (i,)),
        out_shape=jax.ShapeDtypeStruct((nodes_pad,), jnp.float32),
    )(Xp, a2, s, t2, deg2)
    return gg_pad[:n_nodes]


# speedup vs baseline: 11.2156x; 1.0297x over previous
import functools

import jax
import jax.numpy as jnp
from jax import lax
from jax.experimental import pallas as pl
from jax.experimental.pallas import tpu as pltpu
from jax.experimental.pallas import tpu_sc as plsc

_NC = 2
_NS = 16
_NW = _NC * _NS
_L = 16

_B = 80
_TCB = 1024


def _sc_body(nodes_pad, n_edges,
             x_hbm, s_hbm, row_hbm, col_hbm, zb_hbm,
             a_out, t_out, deg_out, *scr):
    idx_rl, idx_cl = scr[0:2]
    idx_r = scr[2:4]
    idx_c = scr[4:6]
    cbuf = scr[6:8]
    sval = scr[8:10]
    ones_v, z1d, a_sp, t_sp, deg_sp = scr[10:15]
    sem_g = scr[15:17]
    sem_s = scr[17:19]
    sem_a, sem_t, sem_d = scr[19:22]

    cid = lax.axis_index("c")
    sid = lax.axis_index("s")
    wid = sid * _NC + cid
    e_per_w = n_edges // _NW
    n_chunks = e_per_w // _B
    per_tile = nodes_pad // _NS

    def _fill2(i, _):
        ones_v[pl.ds(i * _L, _L)] = jnp.ones((_L,), jnp.float32)
        return 0
    lax.fori_loop(0, (_B + _L - 1) // _L, _fill2, 0, unroll=4)

    def _fill1(i, _):
        z1d[pl.ds(i * _L, _L)] = jnp.zeros((_L,), jnp.float32)
        return 0
    lax.fori_loop(0, per_tile // _L, _fill1, 0, unroll=4)

    pltpu.sync_copy(zb_hbm, a_sp.at[pl.ds(sid * per_tile, per_tile)])
    pltpu.sync_copy(z1d, t_sp.at[pl.ds(sid * per_tile, per_tile)])
    pltpu.sync_copy(z1d, deg_sp.at[pl.ds(sid * per_tile, per_tile)])
    plsc.subcore_barrier()

    base_w = wid * e_per_w
    pltpu.sync_copy(row_hbm.at[pl.ds(base_w, e_per_w)], idx_rl)
    pltpu.sync_copy(col_hbm.at[pl.ds(base_w, e_per_w)], idx_cl)

    def _load_idx(b, ci):
        def _mv(k, _):
            sl_d = pl.ds(k * _L, _L)
            sl_s = pl.ds(ci * _B + k * _L, _L)
            idx_r[b][sl_d] = idx_rl[sl_s]
            idx_c[b][sl_d] = idx_cl[sl_s]
            return 0
        lax.fori_loop(0, _B // _L, _mv, 0, unroll=_B // _L)

    def _issue_gathers(b):
        pltpu.async_copy(x_hbm.at[idx_c[b]], cbuf[b], sem_g[b])
        pltpu.async_copy(s_hbm.at[idx_c[b]], sval[b], sem_s[b])

    def _finish_chunk(b):
        pltpu.make_async_copy(x_hbm.at[idx_c[b]], cbuf[b], sem_g[b]).wait()
        pltpu.make_async_copy(s_hbm.at[idx_c[b]], sval[b], sem_s[b]).wait()
        cp_a = pltpu.async_copy(cbuf[b], a_sp.at[idx_r[b]], sem_a, add=True)
        cp_t = pltpu.async_copy(sval[b], t_sp.at[idx_r[b]], sem_t, add=True)
        cp_d = pltpu.async_copy(ones_v, deg_sp.at[idx_r[b]], sem_d,
                                add=True)
        cp_a.wait()
        cp_t.wait()
        cp_d.wait()

    _load_idx(0, 0)
    _issue_gathers(0)

    def _pair(g, _):
        for b in range(2):
            ci = g * 2 + b
            bn = 1 - b

            @pl.when(ci + 1 < n_chunks)
            def _():
                _load_idx(bn, ci + 1)
                _issue_gathers(bn)

            _finish_chunk(b)
        return 0

    lax.fori_loop(0, n_chunks // 2, _pair, 0)
    if n_chunks % 2:
        _finish_chunk((n_chunks - 1) % 2)
    plsc.subcore_barrier()

    sl = pl.ds(sid * per_tile, per_tile)
    pltpu.sync_copy(a_sp.at[sl], a_out.at[cid, sl])
    pltpu.sync_copy(t_sp.at[sl], t_out.at[cid, sl])
    pltpu.sync_copy(deg_sp.at[sl], deg_out.at[cid, sl])


def _tc_rowsumsq(x_ref, s_ref):
    x = x_ref[...]
    s_ref[...] = jnp.sum(x * x, axis=1)


def _tc_epilogue(x_ref, a_ref, s_ref, t_ref, d_ref, out_ref):
    a = a_ref[0] + a_ref[1]
    dot = jnp.sum(x_ref[...] * a, axis=1)
    t = t_ref[0] + t_ref[1]
    dg = d_ref[0] + d_ref[1]
    sums = dg * s_ref[...] + t - 2.0 * dot
    mean = jnp.where(dg > 0.0, sums / jnp.maximum(dg, 1.0), 0.0)
    out_ref[...] = jnp.tanh(mean)


def kernel(X, edge_index, W):
    del W
    n_nodes, d_feat = X.shape
    n_edges = edge_index.shape[1]
    nodes_pad = ((n_nodes + _NS * _L - 1) // (_NS * _L)) * (_NS * _L)

    Xp = jnp.zeros((nodes_pad, d_feat), X.dtype).at[:n_nodes].set(X)
    grid = nodes_pad // _TCB

    s = pl.pallas_call(
        _tc_rowsumsq,
        grid=(grid,),
        in_specs=[pl.BlockSpec((_TCB, d_feat), lambda i: (i, 0))],
        out_specs=pl.BlockSpec((_TCB,), lambda i: (i,)),
        out_shape=jax.ShapeDtypeStruct((nodes_pad,), jnp.float32),
    )(Xp)

    mesh = plsc.VectorSubcoreMesh(core_axis_name="c", subcore_axis_name="s")
    sc = pl.kernel(
        functools.partial(_sc_body, nodes_pad, n_edges),
        mesh=mesh,
        out_type=(
            jax.ShapeDtypeStruct((_NC, nodes_pad, d_feat), jnp.float32),
            jax.ShapeDtypeStruct((_NC, nodes_pad), jnp.float32),
            jax.ShapeDtypeStruct((_NC, nodes_pad), jnp.float32),
        ),
        scratch_types=[pltpu.VMEM((n_edges // _NW,), jnp.int32)
                       for _ in range(2)]
          + [pltpu.VMEM((_B,), jnp.int32) for _ in range(2)]
          + [pltpu.VMEM((_B,), jnp.int32) for _ in range(2)]
          + [pltpu.VMEM((_B, d_feat), jnp.float32) for _ in range(2)]
          + [pltpu.VMEM((_B,), jnp.float32) for _ in range(2)]
          + [
            pltpu.VMEM((((_B + _L - 1) // _L) * _L,), jnp.float32),
            pltpu.VMEM((nodes_pad // _NS,), jnp.float32),
            pltpu.VMEM_SHARED((nodes_pad, d_feat), jnp.float32),
            pltpu.VMEM_SHARED((nodes_pad,), jnp.float32),
            pltpu.VMEM_SHARED((nodes_pad,), jnp.float32),
        ] + [pltpu.SemaphoreType.DMA for _ in range(7)],
    )
    zb = jnp.zeros((nodes_pad // _NS, d_feat), jnp.float32)
    a2, t2, deg2 = sc(Xp, s, edge_index[0], edge_index[1], zb)

    gg_pad = pl.pallas_call(
        _tc_epilogue,
        grid=(grid,),
        in_specs=[
            pl.BlockSpec((_TCB, d_feat), lambda i: (i, 0)),
            pl.BlockSpec((_NC, _TCB, d_feat), lambda i: (0, i, 0)),
            pl.BlockSpec((_TCB,), lambda i: (i,)),
            pl.BlockSpec((_NC, _TCB), lambda i: (0, i)),
            pl.BlockSpec((_NC, _TCB), lambda i: (0, i)),
        ],
        out_specs=pl.BlockSpec((_TCB,), lambda i: (i,)),
        out_shape=jax.ShapeDtypeStruct((nodes_pad,), jnp.float32),
    )(Xp, a2, s, t2, deg2)
    return gg_pad[:n_nodes]


# speedup vs baseline: 11.4288x; 1.0190x over previous
import functools

import jax
import jax.numpy as jnp
from jax import lax
from jax.experimental import pallas as pl
from jax.experimental.pallas import tpu as pltpu
from jax.experimental.pallas import tpu_sc as plsc

_NC = 2
_NS = 16
_NW = _NC * _NS
_L = 16

_B = 80


def _part_chunks(n_chunks):
    p = min(25, n_chunks)
    while n_chunks % p:
        p -= 1
    return p
_TCB = 1024


def _sc_body(nodes_pad, n_edges,
             x_hbm, s_hbm, row_hbm, col_hbm, zb_hbm,
             a_out, t_out, deg_out, *scr):
    idx_rl, idx_cl = scr[0:2]
    idx_r = scr[2:5]
    idx_c = scr[5:8]
    cbuf = scr[8:11]
    sval = scr[11:14]
    ones_v, z1d, a_sp, t_sp, deg_sp = scr[14:19]
    sem_g = scr[19:22]
    sem_s = scr[22:25]
    sem_a = scr[25:28]
    sem_t = scr[28:31]
    sem_d = scr[31:34]

    cid = lax.axis_index("c")
    sid = lax.axis_index("s")
    wid = sid * _NC + cid
    e_per_w = n_edges // _NW
    n_chunks = e_per_w // _B
    per_tile = nodes_pad // _NS

    def _fill2(i, _):
        ones_v[pl.ds(i * _L, _L)] = jnp.ones((_L,), jnp.float32)
        return 0
    lax.fori_loop(0, (_B + _L - 1) // _L, _fill2, 0, unroll=4)

    def _fill1(i, _):
        z1d[pl.ds(i * _L, _L)] = jnp.zeros((_L,), jnp.float32)
        return 0
    lax.fori_loop(0, per_tile // _L, _fill1, 0, unroll=4)

    pltpu.sync_copy(zb_hbm, a_sp.at[pl.ds(sid * per_tile, per_tile)])
    pltpu.sync_copy(z1d, t_sp.at[pl.ds(sid * per_tile, per_tile)])
    pltpu.sync_copy(z1d, deg_sp.at[pl.ds(sid * per_tile, per_tile)])
    plsc.subcore_barrier()

    base_w = wid * e_per_w
    part = _part_chunks(n_chunks)

    def _stage_part(first_ci):
        src = pl.ds(base_w + first_ci * _B, part * _B)
        pltpu.sync_copy(row_hbm.at[src], idx_rl)
        pltpu.sync_copy(col_hbm.at[src], idx_cl)

    def _maybe_restage(nxt):
        if isinstance(nxt, int):
            if nxt % part == 0:
                _stage_part(nxt)
        else:
            @pl.when(nxt % part == 0)
            def _():
                _stage_part(nxt)

    def _load_idx(b, ci):
        off = (ci % part) * _B

        def _mv(k, _):
            sl_d = pl.ds(k * _L, _L)
            sl_s = pl.ds(off + k * _L, _L)
            idx_r[b][sl_d] = idx_rl[sl_s]
            idx_c[b][sl_d] = idx_cl[sl_s]
            return 0
        lax.fori_loop(0, _B // _L, _mv, 0, unroll=_B // _L)

    def _issue_gathers(b):
        pltpu.async_copy(x_hbm.at[idx_c[b]], cbuf[b], sem_g[b])
        pltpu.async_copy(s_hbm.at[idx_c[b]], sval[b], sem_s[b])

    def _wait_gathers(b):
        pltpu.make_async_copy(x_hbm.at[idx_c[b]], cbuf[b], sem_g[b]).wait()
        pltpu.make_async_copy(s_hbm.at[idx_c[b]], sval[b], sem_s[b]).wait()

    def _issue_scatters(b):
        pltpu.async_copy(cbuf[b], a_sp.at[idx_r[b]], sem_a[b], add=True)
        pltpu.async_copy(sval[b], t_sp.at[idx_r[b]], sem_t[b], add=True)
        pltpu.async_copy(ones_v, deg_sp.at[idx_r[b]], sem_d[b], add=True)

    def _drain_scatters(b):
        pltpu.make_async_copy(cbuf[b], a_sp.at[idx_r[b]], sem_a[b]).wait()
        pltpu.make_async_copy(sval[b], t_sp.at[idx_r[b]], sem_t[b]).wait()
        pltpu.make_async_copy(ones_v, deg_sp.at[idx_r[b]], sem_d[b]).wait()

    def _step(r, ci, drain_prev2, has_next):
        rn = (r + 1) % 3

        def _advance():
            _maybe_restage(ci + 1)
            _load_idx(rn, ci + 1)
            _issue_gathers(rn)

        if drain_prev2 is None:
            @pl.when(ci >= 2)
            def _():
                _drain_scatters(rn)
        elif drain_prev2:
            _drain_scatters(rn)
        if has_next is None:
            @pl.when(ci + 1 < n_chunks)
            def _():
                _advance()
        elif has_next:
            _advance()
        _wait_gathers(r)
        _issue_scatters(r)

    _stage_part(0)
    _load_idx(0, 0)
    _issue_gathers(0)

    n_groups = n_chunks // 3

    def _group(g, _):
        for r in range(3):
            _step(r, g * 3 + r, drain_prev2=None, has_next=None)
        return 0

    lax.fori_loop(0, n_groups, _group, 0)
    for ci in range(n_groups * 3, n_chunks):
        _step(ci % 3, ci, drain_prev2=ci >= 2, has_next=ci + 1 < n_chunks)
    for ci in range(max(n_chunks - 2, 0), n_chunks):
        _drain_scatters(ci % 3)
    plsc.subcore_barrier()

    sl = pl.ds(sid * per_tile, per_tile)
    pltpu.sync_copy(a_sp.at[sl], a_out.at[cid, sl])
    pltpu.sync_copy(t_sp.at[sl], t_out.at[cid, sl])
    pltpu.sync_copy(deg_sp.at[sl], deg_out.at[cid, sl])


def _tc_rowsumsq(x_ref, s_ref):
    x = x_ref[...]
    s_ref[...] = jnp.sum(x * x, axis=1)


def _tc_epilogue(x_ref, a_ref, s_ref, t_ref, d_ref, out_ref):
    a = a_ref[0] + a_ref[1]
    dot = jnp.sum(x_ref[...] * a, axis=1)
    t = t_ref[0] + t_ref[1]
    dg = d_ref[0] + d_ref[1]
    sums = dg * s_ref[...] + t - 2.0 * dot
    mean = jnp.where(dg > 0.0, sums / jnp.maximum(dg, 1.0), 0.0)
    out_ref[...] = jnp.tanh(mean)


def kernel(X, edge_index, W):
    del W
    n_nodes, d_feat = X.shape
    n_edges = edge_index.shape[1]
    nodes_pad = ((n_nodes + _NS * _L - 1) // (_NS * _L)) * (_NS * _L)

    Xp = jnp.zeros((nodes_pad, d_feat), X.dtype).at[:n_nodes].set(X)
    grid = nodes_pad // _TCB

    s = pl.pallas_call(
        _tc_rowsumsq,
        grid=(grid,),
        in_specs=[pl.BlockSpec((_TCB, d_feat), lambda i: (i, 0))],
        out_specs=pl.BlockSpec((_TCB,), lambda i: (i,)),
        out_shape=jax.ShapeDtypeStruct((nodes_pad,), jnp.float32),
    )(Xp)

    mesh = plsc.VectorSubcoreMesh(core_axis_name="c", subcore_axis_name="s")
    sc = pl.kernel(
        functools.partial(_sc_body, nodes_pad, n_edges),
        mesh=mesh,
        out_type=(
            jax.ShapeDtypeStruct((_NC, nodes_pad, d_feat), jnp.float32),
            jax.ShapeDtypeStruct((_NC, nodes_pad), jnp.float32),
            jax.ShapeDtypeStruct((_NC, nodes_pad), jnp.float32),
        ),
        scratch_types=[
            pltpu.VMEM(
                (_part_chunks(n_edges // _NW // _B) * _B,), jnp.int32)
            for _ in range(2)]
          + [pltpu.VMEM((_B,), jnp.int32) for _ in range(3)]
          + [pltpu.VMEM((_B,), jnp.int32) for _ in range(3)]
          + [pltpu.VMEM((_B, d_feat), jnp.float32) for _ in range(3)]
          + [pltpu.VMEM((_B,), jnp.float32) for _ in range(3)]
          + [
            pltpu.VMEM((((_B + _L - 1) // _L) * _L,), jnp.float32),
            pltpu.VMEM((nodes_pad // _NS,), jnp.float32),
            pltpu.VMEM_SHARED((nodes_pad, d_feat), jnp.float32),
            pltpu.VMEM_SHARED((nodes_pad,), jnp.float32),
            pltpu.VMEM_SHARED((nodes_pad,), jnp.float32),
        ] + [pltpu.SemaphoreType.DMA for _ in range(15)],
    )
    zb = jnp.zeros((nodes_pad // _NS, d_feat), jnp.float32)
    a2, t2, deg2 = sc(Xp, s, edge_index[0], edge_index[1], zb)

    gg_pad = pl.pallas_call(
        _tc_epilogue,
        grid=(grid,),
        in_specs=[
            pl.BlockSpec((_TCB, d_feat), lambda i: (i, 0)),
            pl.BlockSpec((_NC, _TCB, d_feat), lambda i: (0, i, 0)),
            pl.BlockSpec((_TCB,), lambda i: (i,)),
            pl.BlockSpec((_NC, _TCB), lambda i: (0, i)),
            pl.BlockSpec((_NC, _TCB), lambda i: (0, i)),
        ],
        out_specs=pl.BlockSpec((_TCB,), lambda i: (i,)),
        out_shape=jax.ShapeDtypeStruct((nodes_pad,), jnp.float32),
    )(Xp, a2, s, t2, deg2)
    return gg_pad[:n_nodes]
